# baseline (speedup 1.0000x reference)
import functools

import jax
import jax.numpy as jnp
from jax import lax
from jax.experimental import pallas as pl
from jax.experimental.pallas import tpu as pltpu


def _tables_kernel(emb0_ref, emb1_ref, inter_ref, w00_ref, w01_ref,
                   w10_ref, w11_ref, rw_ref, rb_ref, tl_ref, *, n_steps):
    s = pl.program_id(0)

    @pl.when(jnp.logical_and(s < n_steps - 1, s % 2 == 0))
    def _t0():
        h0 = jnp.tanh(lax.dot_general(emb0_ref[...], w00_ref[...],
                                      (((1,), (1,)), ((), ())),
                                      preferred_element_type=jnp.float32))
        t0 = lax.dot_general(h0, w01_ref[...], (((1,), (1,)), ((), ())),
                             preferred_element_type=jnp.float32)
        tl_ref[...] = jnp.concatenate([t0, jnp.zeros_like(t0)], axis=1)

    @pl.when(s % 2 == 1)
    def _t1():
        h1 = jnp.tanh(lax.dot_general(emb1_ref[...], w10_ref[...],
                                      (((1,), (1,)), ((), ())),
                                      preferred_element_type=jnp.float32))
        t1 = lax.dot_general(h1, w11_ref[...], (((1,), (1,)), ((), ())),
                             preferred_element_type=jnp.float32)
        f = jnp.tanh(t1)
        recon = lax.dot_general(f, rw_ref[...], (((1,), (1,)), ((), ())),
                                preferred_element_type=jnp.float32) + rb_ref[...]
        d = inter_ref[...].astype(jnp.float32) - recon
        lrow = jnp.mean(d * d, axis=-1, keepdims=True)
        lane = lax.broadcasted_iota(jnp.int32, t1.shape, 1)
        chunk2 = jnp.where(lane == 0, lrow,
                           jnp.where(lane == 1, jnp.float32(1.0),
                                     jnp.float32(0.0)))
        tl_ref[...] = jnp.concatenate([t1, chunk2], axis=1)

    @pl.when(s == n_steps - 1)
    def _zeros():
        tl_ref[...] = jnp.zeros_like(tl_ref)


def _gather_kernel(x_sref, tl_ref, out_ref, acc_ref, *, blk, d, zrow):
    base = pl.program_id(0) * blk
    sub = lax.broadcasted_iota(jnp.int32, (8, d), 0)
    acc = jnp.zeros((8, d), jnp.float32)
    for g8 in range(blk // 8):
        tile = jnp.zeros((8, d), jnp.float32)
        for j in range(8):
            v = x_sref[base + g8 * 8 + j]
            vi = jnp.where(v == 0, zrow, v - 1)
            c = pl.multiple_of((vi >> 3) << 3, 8)
            chunk = tl_ref[pl.ds(c, 8), :]
            r = vi & 7
            fin = pltpu.roll(chunk[:, :d], j - r, axis=0)
            tile = tile + jnp.where(sub == j, fin, 0.0)
            acc = acc + jnp.where(sub == r, chunk[:, d:], 0.0)
        out_ref[pl.ds(g8 * 8, 8), :] = tile
    acc_ref[...] = acc


def kernel(x, emb0, emb1, inter_initial,
           ae0_w0, ae0_w1, ae0_rb0, ae0_rb1,
           ae1_w0, ae1_w1, ae1_rb0, ae1_rb1,
           rec0_w, rec0_b, rec1_w, rec1_b):
    B = x.shape[0]
    C0, K = emb0.shape
    C1 = emb1.shape[0]
    D = ae0_w1.shape[0]
    span = rec0_w.shape[0]

    TBLK = min(256, C1)
    nb0 = C0 // TBLK
    nb1 = C1 // TBLK
    n_steps = nb0 + nb1 + 1
    n_tab = C0 + C1 + TBLK

    def _m0(s):
        return jnp.minimum(s // 2, nb0 - 1)

    def _m1(s):
        return jnp.clip((s - 1) // 2, 0, nb1 - 1)

    def _mo(s):
        return jnp.where(s == n_steps - 1, n_steps - 1,
                         jnp.where(s % 2 == 0, s // 2, nb0 + s // 2))

    tl = pl.pallas_call(
        functools.partial(_tables_kernel, n_steps=n_steps),
        grid=(n_steps,),
        in_specs=[
            pl.BlockSpec((TBLK, K), lambda s: (_m0(s), 0)),
            pl.BlockSpec((TBLK, K), lambda s: (_m1(s), 0)),
            pl.BlockSpec((TBLK, span), lambda s: (C0 // TBLK + _m1(s), 0)),
            pl.BlockSpec((D, K), lambda s: (0, 0)),
            pl.BlockSpec((D, D), lambda s: (0, 0)),
            pl.BlockSpec((D, K), lambda s: (0, 0)),
            pl.BlockSpec((D, D), lambda s: (0, 0)),
            pl.BlockSpec((span, D), lambda s: (0, 0)),
            pl.BlockSpec((1, span), lambda s: (0, 0)),
        ],
        out_shape=jax.ShapeDtypeStruct((n_tab, 2 * D), jnp.float32),
        out_specs=pl.BlockSpec((TBLK, 2 * D), lambda s: (_mo(s), 0)),
        compiler_params=pltpu.CompilerParams(
            dimension_semantics=("parallel",)),
    )(emb0, emb1, inter_initial, ae0_w0, ae0_w1, ae1_w0, ae1_w1,
      rec0_w, rec0_b.reshape(1, span))

    BLK = min(256, B)
    grid2 = B // BLK
    grid_spec = pltpu.PrefetchScalarGridSpec(
        num_scalar_prefetch=1,
        grid=(grid2,),
        in_specs=[pl.BlockSpec((n_tab, 2 * D), lambda g, xs: (0, 0))],
        out_specs=[pl.BlockSpec((BLK, D), lambda g, xs: (g, 0)),
                   pl.BlockSpec((8, D), lambda g, xs: (g, 0))],
    )
    final, accs = pl.pallas_call(
        functools.partial(_gather_kernel, blk=BLK, d=D, zrow=C0 + C1),
        grid_spec=grid_spec,
        out_shape=(jax.ShapeDtypeStruct((B, D), jnp.float32),
                   jax.ShapeDtypeStruct((grid2 * 8, D), jnp.float32)),
        compiler_params=pltpu.CompilerParams(
            dimension_semantics=("parallel",)),
    )(x, tl)

    lsum = jnp.sum(accs[:, 0])
    cnt = jnp.sum(accs[:, 1])
    loss = jnp.where(cnt > 0, lsum / jnp.maximum(cnt, 1.0), 0.0) * 100.0
    return final, jnp.reshape(loss, (1,))


# speedup vs baseline: 1.3208x; 1.3208x over previous
import functools

import jax
import jax.numpy as jnp
from jax import lax
from jax.experimental import pallas as pl
from jax.experimental.pallas import tpu as pltpu


def _tables_kernel(emb0_ref, emb1_ref, inter_ref, w00_ref, w01_ref,
                   w10_ref, w11_ref, rw_ref, rb_ref, tl_ref, *, n_steps):
    s = pl.program_id(0)

    @pl.when(jnp.logical_and(s < n_steps - 1, s % 2 == 0))
    def _t0():
        h0 = jnp.tanh(lax.dot_general(emb0_ref[...], w00_ref[...],
                                      (((1,), (1,)), ((), ())),
                                      preferred_element_type=jnp.float32))
        t0 = lax.dot_general(h0, w01_ref[...], (((1,), (1,)), ((), ())),
                             preferred_element_type=jnp.float32)
        tl_ref[...] = jnp.concatenate([t0, jnp.zeros_like(t0)], axis=1)

    @pl.when(s % 2 == 1)
    def _t1():
        h1 = jnp.tanh(lax.dot_general(emb1_ref[...], w10_ref[...],
                                      (((1,), (1,)), ((), ())),
                                      preferred_element_type=jnp.float32))
        t1 = lax.dot_general(h1, w11_ref[...], (((1,), (1,)), ((), ())),
                             preferred_element_type=jnp.float32)
        f = jnp.tanh(t1)
        recon = lax.dot_general(f, rw_ref[...], (((1,), (1,)), ((), ())),
                                preferred_element_type=jnp.float32) + rb_ref[...]
        d = inter_ref[...].astype(jnp.float32) - recon
        lrow = jnp.mean(d * d, axis=-1, keepdims=True)
        lane = lax.broadcasted_iota(jnp.int32, t1.shape, 1)
        chunk2 = jnp.where(lane == 0, lrow,
                           jnp.where(lane == 1, jnp.float32(1.0),
                                     jnp.float32(0.0)))
        tl_ref[...] = jnp.concatenate([t1, chunk2], axis=1)

    @pl.when(s == n_steps - 1)
    def _zeros():
        tl_ref[...] = jnp.zeros_like(tl_ref)


def _gather_kernel(c_sref, amt_sref, tl_ref, out_ref, acc_ref, *, blk, d):
    base = pl.program_id(0) * blk
    sub = lax.broadcasted_iota(jnp.int32, (8, d), 0)
    acc = jnp.zeros((8, d), jnp.float32)
    for g8 in range(blk // 8):
        tile = jnp.zeros((8, d), jnp.float32)
        for j in range(8):
            i = base + g8 * 8 + j
            c = pl.multiple_of(c_sref[i], 8)
            chunk = tl_ref[pl.ds(c, 8), :]
            rolled = pltpu.roll(chunk, amt_sref[i], axis=0)
            tile = tile + jnp.where(sub == j, rolled[:, :d], 0.0)
            acc = acc + jnp.where(sub == j, rolled[:, d:], 0.0)
        out_ref[pl.ds(g8 * 8, 8), :] = tile
    acc_ref[...] = acc


def kernel(x, emb0, emb1, inter_initial,
           ae0_w0, ae0_w1, ae0_rb0, ae0_rb1,
           ae1_w0, ae1_w1, ae1_rb0, ae1_rb1,
           rec0_w, rec0_b, rec1_w, rec1_b):
    B = x.shape[0]
    C0, K = emb0.shape
    C1 = emb1.shape[0]
    D = ae0_w1.shape[0]
    span = rec0_w.shape[0]

    TBLK = min(512, C1)
    nb0 = C0 // TBLK
    nb1 = C1 // TBLK
    n_steps = nb0 + nb1 + 1
    n_tab = C0 + C1 + TBLK

    def _m0(s):
        return jnp.minimum(s // 2, nb0 - 1)

    def _m1(s):
        return jnp.clip((s - 1) // 2, 0, nb1 - 1)

    def _mo(s):
        return jnp.where(s == n_steps - 1, n_steps - 1,
                         jnp.where(s % 2 == 0, s // 2, nb0 + s // 2))

    tl = pl.pallas_call(
        functools.partial(_tables_kernel, n_steps=n_steps),
        grid=(n_steps,),
        in_specs=[
            pl.BlockSpec((TBLK, K), lambda s: (_m0(s), 0)),
            pl.BlockSpec((TBLK, K), lambda s: (_m1(s), 0)),
            pl.BlockSpec((TBLK, span), lambda s: (C0 // TBLK + _m1(s), 0)),
            pl.BlockSpec((D, K), lambda s: (0, 0)),
            pl.BlockSpec((D, D), lambda s: (0, 0)),
            pl.BlockSpec((D, K), lambda s: (0, 0)),
            pl.BlockSpec((D, D), lambda s: (0, 0)),
            pl.BlockSpec((span, D), lambda s: (0, 0)),
            pl.BlockSpec((1, span), lambda s: (0, 0)),
        ],
        out_shape=jax.ShapeDtypeStruct((n_tab, 2 * D), jnp.float32),
        out_specs=pl.BlockSpec((TBLK, 2 * D), lambda s: (_mo(s), 0)),
        compiler_params=pltpu.CompilerParams(
            dimension_semantics=("parallel",)),
    )(emb0, emb1, inter_initial, ae0_w0, ae0_w1, ae1_w0, ae1_w1,
      rec0_w, rec0_b.reshape(1, span))

    vi = jnp.where(x == 0, C0 + C1, x - 1)
    c_arr = (vi >> 3) << 3
    amt_arr = (jnp.arange(B, dtype=jnp.int32) & 7) - (vi & 7)

    BLK = min(512, B)
    grid2 = B // BLK
    grid_spec = pltpu.PrefetchScalarGridSpec(
        num_scalar_prefetch=2,
        grid=(grid2,),
        in_specs=[pl.BlockSpec((n_tab, 2 * D), lambda g, cs, ams: (0, 0))],
        out_specs=[pl.BlockSpec((BLK, D), lambda g, cs, ams: (g, 0)),
                   pl.BlockSpec((8, D), lambda g, cs, ams: (g, 0))],
    )
    final, accs = pl.pallas_call(
        functools.partial(_gather_kernel, blk=BLK, d=D),
        grid_spec=grid_spec,
        out_shape=(jax.ShapeDtypeStruct((B, D), jnp.float32),
                   jax.ShapeDtypeStruct((grid2 * 8, D), jnp.float32)),
        compiler_params=pltpu.CompilerParams(
            dimension_semantics=("parallel",)),
    )(c_arr, amt_arr, tl)

    lsum = jnp.sum(accs[:, 0])
    cnt = jnp.sum(accs[:, 1])
    loss = jnp.where(cnt > 0, lsum / jnp.maximum(cnt, 1.0), 0.0) * 100.0
    return final, jnp.reshape(loss, (1,))
